# baseline (speedup 1.0000x reference)
import jax
import jax.numpy as jnp
from jax import lax
from jax.experimental import pallas as pl
from jax.experimental.pallas import tpu as pltpu
from jax.experimental.pallas import tpu_sc as plsc

_NC, _NS = 2, 16
_NW = _NC * _NS
_W_ROWS = 3128
_CHUNK = 128
_TAIL = 8
_REL_CHUNK = 200


def _sc_copy(ent_in, rel_in, ent_out, rel_out, buf, rbuf):
    c = lax.axis_index("c")
    s = lax.axis_index("s")
    w = s * _NC + c
    n = ent_in.shape[0]
    base = w * _W_ROWS
    cnt = jnp.minimum(_W_ROWS, n - base)
    nfull = cnt // _CHUNK

    def body(j, carry):
        start = base + j * _CHUNK
        pltpu.sync_copy(ent_in.at[pl.ds(start, _CHUNK)], buf)
        pltpu.sync_copy(buf, ent_out.at[pl.ds(start, _CHUNK)])
        return carry

    lax.fori_loop(0, nfull, body, 0)

    tail_base = base + nfull * _CHUNK
    ntail = (cnt - nfull * _CHUNK) // _TAIL

    def tbody(j, carry):
        start = tail_base + j * _TAIL
        pltpu.sync_copy(ent_in.at[pl.ds(start, _TAIL)],
                        buf.at[pl.ds(0, _TAIL)])
        pltpu.sync_copy(buf.at[pl.ds(0, _TAIL)],
                        ent_out.at[pl.ds(start, _TAIL)])
        return carry

    lax.fori_loop(0, ntail, tbody, 0)

    @pl.when(w < 5)
    def _():
        start = w * _REL_CHUNK
        pltpu.sync_copy(rel_in.at[pl.ds(start, _REL_CHUNK)], rbuf)
        pltpu.sync_copy(rbuf, rel_out.at[pl.ds(start, _REL_CHUNK)])


def kernel(emb_ent, emb_rel, edge_index, rel, edge_index_all, rel_all):
    return tuple(pl.kernel(
        _sc_copy,
        out_type=[jax.ShapeDtypeStruct(emb_ent.shape, emb_ent.dtype),
                  jax.ShapeDtypeStruct(emb_rel.shape, emb_rel.dtype)],
        mesh=plsc.VectorSubcoreMesh(core_axis_name="c", subcore_axis_name="s"),
        scratch_types=[pltpu.VMEM((_CHUNK, 768), jnp.float32),
                       pltpu.VMEM((_REL_CHUNK, 128), jnp.float32)],
    )(emb_ent, emb_rel))


# speedup vs baseline: 1.1695x; 1.1695x over previous
import jax
import jax.numpy as jnp
from jax import lax
from jax.experimental import pallas as pl
from jax.experimental.pallas import tpu as pltpu
from jax.experimental.pallas import tpu_sc as plsc

_ENT_BLOCK = 4000
_NC = 2
_REL_CHUNK = 200


def _copy_block(src_ref, dst_ref):
    dst_ref[...] = src_ref[...]


def _sc_rel_copy(rel_in, rel_out, rbuf):
    c = lax.axis_index("c")
    s = lax.axis_index("s")
    w = s * _NC + c

    @pl.when(w < 5)
    def _():
        start = w * _REL_CHUNK
        pltpu.sync_copy(rel_in.at[pl.ds(start, _REL_CHUNK)], rbuf)
        pltpu.sync_copy(rbuf, rel_out.at[pl.ds(start, _REL_CHUNK)])


def kernel(emb_ent, emb_rel, edge_index, rel, edge_index_all, rel_all):
    rel_out = pl.kernel(
        _sc_rel_copy,
        out_type=jax.ShapeDtypeStruct(emb_rel.shape, emb_rel.dtype),
        mesh=plsc.VectorSubcoreMesh(core_axis_name="c", subcore_axis_name="s"),
        scratch_types=[pltpu.VMEM((_REL_CHUNK, 128), jnp.float32)],
    )(emb_rel)
    n, d = emb_ent.shape
    ent_out = pl.pallas_call(
        _copy_block,
        grid=(n // _ENT_BLOCK,),
        in_specs=[pl.BlockSpec((_ENT_BLOCK, d), lambda i: (i, 0))],
        out_specs=pl.BlockSpec((_ENT_BLOCK, d), lambda i: (i, 0)),
        out_shape=jax.ShapeDtypeStruct((n, d), emb_ent.dtype),
        compiler_params=pltpu.CompilerParams(
            dimension_semantics=("arbitrary",)),
    )(emb_ent)
    return (ent_out, rel_out)


# speedup vs baseline: 1.2605x; 1.0778x over previous
import jax
import jax.numpy as jnp
from jax.experimental import pallas as pl
from jax.experimental.pallas import tpu as pltpu

_CHUNK = 1000
_NCHUNK = 100
_NBUF = 16
_LA = 8


def _ring_copy(ent_in, rel_in, ent_out, rel_out, buf, rbuf,
               lsem, ssem, rlsem, rssem):
    rel_load = pltpu.make_async_copy(rel_in, rbuf, rlsem)
    rel_store = pltpu.make_async_copy(rbuf, rel_out, rssem)
    rel_load.start()

    def load(j):
        sl = pl.ds(j * _CHUNK, _CHUNK)
        return pltpu.make_async_copy(ent_in.at[sl], buf.at[j % _NBUF],
                                     lsem.at[j % _NBUF])

    def store(j):
        sl = pl.ds(j * _CHUNK, _CHUNK)
        return pltpu.make_async_copy(buf.at[j % _NBUF], ent_out.at[sl],
                                     ssem.at[j % _NBUF])

    for j in range(_LA):
        load(j).start()

    rel_load.wait()
    rel_store.start()

    for i in range(_NCHUNK):
        load(i).wait()
        store(i).start()
        j = i + _LA
        if j < _NCHUNK:
            if j >= _NBUF:
                store(j - _NBUF).wait()
            load(j).start()

    for k in range(_NCHUNK - _NBUF, _NCHUNK):
        store(k).wait()
    rel_store.wait()


def kernel(emb_ent, emb_rel, edge_index, rel, edge_index_all, rel_all):
    return tuple(pl.pallas_call(
        _ring_copy,
        in_specs=[pl.BlockSpec(memory_space=pl.ANY),
                  pl.BlockSpec(memory_space=pl.ANY)],
        out_specs=[pl.BlockSpec(memory_space=pl.ANY),
                   pl.BlockSpec(memory_space=pl.ANY)],
        out_shape=[jax.ShapeDtypeStruct(emb_ent.shape, emb_ent.dtype),
                   jax.ShapeDtypeStruct(emb_rel.shape, emb_rel.dtype)],
        scratch_shapes=[
            pltpu.VMEM((_NBUF, _CHUNK, 768), jnp.float32),
            pltpu.VMEM(emb_rel.shape, jnp.float32),
            pltpu.SemaphoreType.DMA((_NBUF,)),
            pltpu.SemaphoreType.DMA((_NBUF,)),
            pltpu.SemaphoreType.DMA,
            pltpu.SemaphoreType.DMA,
        ],
    )(emb_ent, emb_rel))


# speedup vs baseline: 1.2644x; 1.0031x over previous
import jax
import jax.numpy as jnp
from jax.experimental import pallas as pl
from jax.experimental.pallas import tpu as pltpu

_CHUNK = 2000
_NCHUNK = 50
_NBUF = 8
_LA = 4


def _ring_copy(ent_in, rel_in, ent_out, rel_out, buf, rbuf,
               lsem, ssem, rlsem, rssem):
    rel_load = pltpu.make_async_copy(rel_in, rbuf, rlsem)
    rel_store = pltpu.make_async_copy(rbuf, rel_out, rssem)
    rel_load.start()

    def load(j):
        sl = pl.ds(j * _CHUNK, _CHUNK)
        return pltpu.make_async_copy(ent_in.at[sl], buf.at[j % _NBUF],
                                     lsem.at[j % _NBUF])

    def store(j):
        sl = pl.ds(j * _CHUNK, _CHUNK)
        return pltpu.make_async_copy(buf.at[j % _NBUF], ent_out.at[sl],
                                     ssem.at[j % _NBUF])

    for j in range(_LA):
        load(j).start()

    rel_load.wait()
    rel_store.start()

    for i in range(_NCHUNK):
        load(i).wait()
        store(i).start()
        j = i + _LA
        if j < _NCHUNK:
            if j >= _NBUF:
                store(j - _NBUF).wait()
            load(j).start()

    for k in range(_NCHUNK - _NBUF, _NCHUNK):
        store(k).wait()
    rel_store.wait()


def kernel(emb_ent, emb_rel, edge_index, rel, edge_index_all, rel_all):
    return tuple(pl.pallas_call(
        _ring_copy,
        in_specs=[pl.BlockSpec(memory_space=pl.ANY),
                  pl.BlockSpec(memory_space=pl.ANY)],
        out_specs=[pl.BlockSpec(memory_space=pl.ANY),
                   pl.BlockSpec(memory_space=pl.ANY)],
        out_shape=[jax.ShapeDtypeStruct(emb_ent.shape, emb_ent.dtype),
                   jax.ShapeDtypeStruct(emb_rel.shape, emb_rel.dtype)],
        scratch_shapes=[
            pltpu.VMEM((_NBUF, _CHUNK, 768), jnp.float32),
            pltpu.VMEM(emb_rel.shape, jnp.float32),
            pltpu.SemaphoreType.DMA((_NBUF,)),
            pltpu.SemaphoreType.DMA((_NBUF,)),
            pltpu.SemaphoreType.DMA,
            pltpu.SemaphoreType.DMA,
        ],
    )(emb_ent, emb_rel))
